# baseline (speedup 1.0000x reference)
import functools

import jax
import jax.numpy as jnp
from jax import lax
from jax.experimental import pallas as pl
from jax.experimental.pallas import tpu as pltpu
from jax.experimental.pallas import tpu_sc as plsc

EMBED = 64
NUM_CORES = 2
NUM_SUBCORES = 16
NUM_WORKERS = NUM_CORES * NUM_SUBCORES
CHUNK = 8


def _make_lookup(batch: int, hist: int):
    rows_per_w = batch // NUM_WORKERS
    n_pairs = rows_per_w // (2 * CHUNK)
    mesh = plsc.VectorSubcoreMesh(core_axis_name="c", subcore_axis_name="s")

    @functools.partial(
        pl.kernel,
        mesh=mesh,
        out_type=jax.ShapeDtypeStruct((batch, hist, EMBED), jnp.float32),
        scratch_types=[
            pltpu.VMEM((rows_per_w, hist), jnp.int32),
            pltpu.VMEM((2, CHUNK, hist, EMBED), jnp.float32),
            pltpu.SemaphoreType.DMA,
            pltpu.SemaphoreType.DMA,
            pltpu.SemaphoreType.DMA,
            pltpu.SemaphoreType.DMA,
        ],
        compiler_params=pltpu.CompilerParams(use_tc_tiling_on_sc=False),
    )
    def lookup(table_hbm, idx_hbm, out_hbm, idx_v, rows_v, gsem_a, gsem_b,
               osem_a, osem_b):
        wid = lax.axis_index("s") * NUM_CORES + lax.axis_index("c")
        base = wid * rows_per_w
        pltpu.sync_copy(idx_hbm.at[pl.ds(base, rows_per_w)], idx_v)

        def gat_each(chunk, slot, sem, op):
            for rr in range(CHUNK):
                getattr(
                    pltpu.make_async_copy(
                        table_hbm.at[idx_v.at[chunk * CHUNK + rr]],
                        rows_v.at[slot, rr],
                        sem,
                    ),
                    op,
                )()

        def out(chunk, slot, sem):
            return pltpu.make_async_copy(
                rows_v.at[slot], out_hbm.at[pl.ds(base + chunk * CHUNK, CHUNK)],
                sem)

        gat_each(0, 0, gsem_a, "start")

        def body(p, carry):
            c0 = 2 * p
            c1 = c0 + 1

            @pl.when(p > 0)
            def _():
                out(c1 - 2, 1, osem_b).wait()

            gat_each(c1, 1, gsem_b, "start")
            gat_each(c0, 0, gsem_a, "wait")
            out(c0, 0, osem_a).start()
            gat_each(c1, 1, gsem_b, "wait")
            out(c1, 1, osem_b).start()

            @pl.when(p < n_pairs - 1)
            def _():
                out(c0, 0, osem_a).wait()
                gat_each(c0 + 2, 0, gsem_a, "start")

            return carry

        lax.fori_loop(0, n_pairs, body, 0)
        out(2 * n_pairs - 2, 0, osem_a).wait()
        out(2 * n_pairs - 1, 1, osem_b).wait()

    return lookup


def kernel(entity_table, type_index):
    batch, hist = type_index.shape
    idx = type_index.astype(jnp.int32)
    return _make_lookup(batch, hist)(entity_table, idx)
